# baseline (speedup 1.0000x reference)
import functools

import jax
import jax.numpy as jnp
from jax import lax
from jax.experimental import pallas as pl
from jax.experimental.pallas import tpu as pltpu
from jax.experimental.pallas import tpu_sc as plsc

VOCAB = 100000
EMB_DIM = 300
PAD_DIM = 304
B, L = 4096, 200
N = B * L

NC, NS = 2, 16
NW = NC * NS
PER_W = N // NW
C = 128
N_CHUNKS = PER_W // C


def _make_sc_gather():
    mesh = plsc.VectorSubcoreMesh(core_axis_name="c", subcore_axis_name="s")

    @functools.partial(
        pl.kernel,
        mesh=mesh,
        compiler_params=pltpu.CompilerParams(use_tc_tiling_on_sc=False),
        out_type=jax.ShapeDtypeStruct((N, PAD_DIM), jnp.float32),
        scratch_types=[
            pltpu.VMEM((N_CHUNKS, C), jnp.int32),
            pltpu.VMEM((C, PAD_DIM), jnp.float32),
            pltpu.SemaphoreType.DMA,
        ],
    )
    def sc_gather(idx_hbm, table_hbm, out_hbm, idx_v, rows, gsem):
        wid = lax.axis_index("s") * NC + lax.axis_index("c")
        base = wid * PER_W
        pltpu.sync_copy(idx_hbm.at[wid], idx_v)

        def body(j, carry):
            pltpu.async_copy(table_hbm.at[idx_v.at[j]], rows, gsem).wait()
            pltpu.sync_copy(rows, out_hbm.at[pl.ds(base + j * C, C)])
            return carry

        lax.fori_loop(0, N_CHUNKS, body, 0)

    return sc_gather


_sc_gather = _make_sc_gather()


def kernel(indices, emb_weight):
    idx = indices.reshape(NW, N_CHUNKS, C).astype(jnp.int32)
    table = jnp.pad(emb_weight, ((0, 0), (0, PAD_DIM - EMB_DIM)))
    out = _sc_gather(idx, table)
    return out[:, :EMB_DIM].reshape(B, L, EMB_DIM)


# speedup vs baseline: 1.4369x; 1.4369x over previous
import functools

import jax
import jax.numpy as jnp
from jax import lax
from jax.experimental import pallas as pl
from jax.experimental.pallas import tpu as pltpu
from jax.experimental.pallas import tpu_sc as plsc

VOCAB = 100000
EMB_DIM = 300
PAD_DIM = 384
BAND = 256
B, L = 4096, 200
N = B * L

NC, NS = 2, 16
NW = NC * NS
PER_W = N // NW
C = 128
N_CHUNKS = PER_W // C


def _make_sc_gather():
    mesh = plsc.VectorSubcoreMesh(core_axis_name="c", subcore_axis_name="s")

    @functools.partial(
        pl.kernel,
        mesh=mesh,
        compiler_params=pltpu.CompilerParams(needs_layout_passes=False),
        out_type=jax.ShapeDtypeStruct((N, EMB_DIM), jnp.float32),
        scratch_types=[
            pltpu.VMEM((N_CHUNKS, C), jnp.int32),
            pltpu.VMEM((C, EMB_DIM), jnp.float32),
            pltpu.VMEM((C, 128), jnp.float32),
            pltpu.SemaphoreType.DMA,
            pltpu.SemaphoreType.DMA,
        ],
    )
    def sc_gather(idx_hbm, table_hbm, out_hbm, idx_v, rows300, tail, gb, gt):
        wid = lax.axis_index("s") * NC + lax.axis_index("c")
        base = wid * PER_W
        pltpu.sync_copy(idx_hbm.at[wid], idx_v)

        band_src = table_hbm.at[:, pl.ds(0, BAND)]
        tail_src = table_hbm.at[:, pl.ds(BAND, 128)]

        lanes = lax.iota(jnp.int32, 16)
        tail_cols = BAND + 32 + lanes
        tail_mask = lanes < (EMB_DIM - BAND - 32)

        def body(j, carry):
            idx_j = idx_v.at[j]
            pltpu.async_copy(band_src.at[idx_j], rows300.at[:, pl.ds(0, BAND)], gb)
            pltpu.async_copy(tail_src.at[idx_j], tail, gt)
            pltpu.make_async_copy(
                band_src.at[idx_j], rows300.at[:, pl.ds(0, BAND)], gb).wait()
            pltpu.make_async_copy(tail_src.at[idx_j], tail, gt).wait()

            def rbody(r, c2):
                rows300[r, pl.ds(BAND, 16)] = tail[r, pl.ds(0, 16)]
                rows300[r, pl.ds(BAND + 16, 16)] = tail[r, pl.ds(16, 16)]
                v2 = tail[r, pl.ds(32, 16)]
                rids = jnp.full((16,), r, jnp.int32)
                plsc.store_scatter(rows300, [rids, tail_cols], v2,
                                   mask=tail_mask)
                return c2

            lax.fori_loop(0, C, rbody, 0)
            pltpu.sync_copy(rows300, out_hbm.at[pl.ds(base + j * C, C)])
            return carry

        lax.fori_loop(0, N_CHUNKS, body, 0)

    return sc_gather


_sc_gather = _make_sc_gather()


def kernel(indices, emb_weight):
    idx = indices.reshape(NW, N_CHUNKS, C).astype(jnp.int32)
    table = jnp.pad(emb_weight, ((0, 0), (0, PAD_DIM - EMB_DIM)))
    out = _sc_gather(idx, table)
    return out.reshape(B, L, EMB_DIM)


# speedup vs baseline: 1.6609x; 1.1559x over previous
import functools

import jax
import jax.numpy as jnp
from jax import lax
from jax.experimental import pallas as pl
from jax.experimental.pallas import tpu as pltpu
from jax.experimental.pallas import tpu_sc as plsc

VOCAB = 100000
EMB_DIM = 300
PAD_DIM = 384
BAND = 256
B, L = 4096, 200
N = B * L

NC, NS = 2, 16
NW = NC * NS
PER_W = N // NW
C = 64
N_CHUNKS = PER_W // C
NB = 2


def _make_sc_gather():
    mesh = plsc.VectorSubcoreMesh(core_axis_name="c", subcore_axis_name="s")

    @functools.partial(
        pl.kernel,
        mesh=mesh,
        compiler_params=pltpu.CompilerParams(needs_layout_passes=False),
        out_type=jax.ShapeDtypeStruct((N, EMB_DIM), jnp.float32),
        scratch_types=[
            pltpu.VMEM((N_CHUNKS, C), jnp.int32),
            pltpu.VMEM((C, EMB_DIM), jnp.float32),
            pltpu.VMEM((C, EMB_DIM), jnp.float32),
            pltpu.VMEM((C, 128), jnp.float32),
            pltpu.VMEM((C, 128), jnp.float32),
            pltpu.SemaphoreType.DMA,
            pltpu.SemaphoreType.DMA,
            pltpu.SemaphoreType.DMA,
            pltpu.SemaphoreType.DMA,
            pltpu.SemaphoreType.DMA,
            pltpu.SemaphoreType.DMA,
        ],
    )
    def sc_gather(idx_hbm, table_hbm, out_hbm, idx_v,
                  rows0, rows1, tail0, tail1,
                  gb0, gb1, gt0, gt1, st0, st1):
        wid = lax.axis_index("s") * NC + lax.axis_index("c")
        base = wid * PER_W
        pltpu.sync_copy(idx_hbm.at[wid], idx_v)

        band_src = table_hbm.at[:, pl.ds(0, BAND)]
        tail_src = table_hbm.at[:, pl.ds(BAND, 128)]

        rows = (rows0, rows1)
        tails = (tail0, tail1)
        gbs = (gb0, gb1)
        gts = (gt0, gt1)
        sts = (st0, st1)

        lanes = lax.iota(jnp.int32, 16)
        tail_cols = BAND + 32 + lanes
        tail_mask = lanes < (EMB_DIM - BAND - 32)

        def fire(j, b):
            idx_j = idx_v.at[j]
            pltpu.async_copy(band_src.at[idx_j],
                             rows[b].at[:, pl.ds(0, BAND)], gbs[b])
            pltpu.async_copy(tail_src.at[idx_j], tails[b], gts[b])

        def wait_gathers(j, b):
            idx_j = idx_v.at[j]
            pltpu.make_async_copy(band_src.at[idx_j],
                                  rows[b].at[:, pl.ds(0, BAND)], gbs[b]).wait()
            pltpu.make_async_copy(tail_src.at[idx_j], tails[b], gts[b]).wait()

        def assemble(b):
            def rbody(r, c2):
                rows[b][r, pl.ds(BAND, 16)] = tails[b][r, pl.ds(0, 16)]
                rows[b][r, pl.ds(BAND + 16, 16)] = tails[b][r, pl.ds(16, 16)]
                v2 = tails[b][r, pl.ds(32, 16)]
                rids = jnp.full((16,), r, jnp.int32)
                plsc.store_scatter(rows[b], [rids, tail_cols], v2,
                                   mask=tail_mask)
                return c2
            lax.fori_loop(0, C, rbody, 0)

        def out_dst(j):
            return out_hbm.at[pl.ds(base + j * C, C)]

        for b in range(NB):
            fire(b, b)

        def body(jo2, carry):
            jo = jo2 * NB
            for b in range(NB):
                j = jo + b
                wait_gathers(j, b)
                assemble(b)
                pltpu.async_copy(rows[b], out_dst(j), sts[b])
            for b in range(NB):
                j = jo + b + NB

                @pl.when(j < N_CHUNKS)
                def _():
                    pltpu.make_async_copy(rows[b], out_dst(j - NB),
                                          sts[b]).wait()
                    fire(j, b)
            return carry

        lax.fori_loop(0, N_CHUNKS // NB, body, 0)

        for b in range(NB):
            j = N_CHUNKS - NB + b
            pltpu.make_async_copy(rows[b], out_dst(j), sts[b]).wait()

    return sc_gather


_sc_gather = _make_sc_gather()


def kernel(indices, emb_weight):
    idx = indices.reshape(NW, N_CHUNKS, C).astype(jnp.int32)
    table = jnp.pad(emb_weight, ((0, 0), (0, PAD_DIM - EMB_DIM)))
    out = _sc_gather(idx, table)
    return out.reshape(B, L, EMB_DIM)
